# baseline (speedup 1.0000x reference)
import functools

import jax
import jax.numpy as jnp
from jax import lax
from jax.experimental import pallas as pl
from jax.experimental.pallas import tpu as pltpu
from jax.experimental.pallas import tpu_sc as plsc

N_GENES = 10000
D = 128
N_EDGES = 320000

NC = 2
NS = 16
L = 16

NPAD = 10240
EDGES_PER_TILE_DEG = N_EDGES // NS
DEG_CHUNK = 2000
AGG_CHUNK = 128
AGG_NCHUNK = 80
AGG_HALF = AGG_NCHUNK // 2
EPAD = NC * NS * AGG_NCHUNK * AGG_CHUNK
TRASH_ROW = N_GENES + 16
NACC = 10112
ACC_PER_TILE = NACC // NS


def _newton_rsqrt(x):
    i = plsc.bitcast(x, jnp.int32)
    y = plsc.bitcast(jnp.int32(0x5F3759DF) - (i >> 1), jnp.float32)
    for _ in range(3):
        y = y * (1.5 - 0.5 * x * y * y)
    return y


def _deg_kernel(dst_hbm, dinv_hbm, dstbuf, deg_tile, tmp, acc, deg_sh):
    cid = lax.axis_index("c")
    sid = lax.axis_index("s")

    @pl.when(cid == 0)
    def _():
        zeros16 = jnp.zeros((L,), jnp.float32)

        def zloop(i, _):
            deg_tile[pl.ds(i * L, L)] = zeros16
            return 0
        lax.fori_loop(0, NPAD // L, zloop, 0)

        ones = zeros16 + 1.0

        def chunk(j, _):
            pltpu.sync_copy(dst_hbm.at[pl.ds(sid * EDGES_PER_TILE_DEG
                                             + j * DEG_CHUNK, DEG_CHUNK)],
                            dstbuf)

            def scat(k, _):
                for u in range(5):
                    idx = dstbuf[pl.ds((k * 5 + u) * L, L)]
                    plsc.addupdate_scatter(deg_tile, [idx], ones)
                return 0
            lax.fori_loop(0, DEG_CHUNK // L // 5, scat, 0)
            return 0
        lax.fori_loop(0, EDGES_PER_TILE_DEG // DEG_CHUNK, chunk, 0)

        pltpu.sync_copy(deg_tile, deg_sh.at[sid])
        plsc.subcore_barrier()

        stripe = NPAD // NS
        def zacc(i, _):
            acc[pl.ds(i * L, L)] = zeros16
            return 0
        lax.fori_loop(0, stripe // L, zacc, 0)

        for t in range(NS):
            pltpu.sync_copy(deg_sh.at[t, pl.ds(sid * stripe, stripe)], tmp)

            def addl(i, _):
                acc[pl.ds(i * L, L)] = acc[pl.ds(i * L, L)] + tmp[pl.ds(i * L, L)]
                return 0
            lax.fori_loop(0, stripe // L, addl, 0)

        def fin(i, _):
            d = acc[pl.ds(i * L, L)] + 1.0
            acc[pl.ds(i * L, L)] = _newton_rsqrt(d)
            return 0
        lax.fori_loop(0, stripe // L, fin, 0)

        pltpu.sync_copy(acc, dinv_hbm.at[pl.ds(sid * stripe, stripe)])


def _deg_call(dst):
    mesh = plsc.VectorSubcoreMesh(core_axis_name="c", subcore_axis_name="s")

    @functools.partial(
        pl.kernel,
        out_type=jax.ShapeDtypeStruct((NPAD,), jnp.float32),
        mesh=mesh,
        scratch_types=[
            pltpu.VMEM((DEG_CHUNK,), jnp.int32),
            pltpu.VMEM((NPAD,), jnp.float32),
            pltpu.VMEM((NPAD // NS,), jnp.float32),
            pltpu.VMEM((NPAD // NS,), jnp.float32),
            pltpu.VMEM_SHARED((NS, NPAD), jnp.float32),
        ],
        compiler_params=pltpu.CompilerParams(needs_layout_passes=False),
    )
    def call(dst_hbm, dinv_hbm, dstbuf, deg_tile, tmp, acc, deg_sh):
        _deg_kernel(dst_hbm, dinv_hbm, dstbuf, deg_tile, tmp, acc, deg_sh)

    return call(dst)


def _agg_call(hs, src2d, dst2d):
    mesh = plsc.VectorSubcoreMesh(core_axis_name="c", subcore_axis_name="s")

    @functools.partial(
        pl.kernel,
        out_type=jax.ShapeDtypeStruct((NC, NACC, D), jnp.float32),
        mesh=mesh,
        scratch_types=[
            pltpu.VMEM((AGG_HALF, AGG_CHUNK), jnp.int32),
            pltpu.VMEM((AGG_HALF, AGG_CHUNK), jnp.int32),
            pltpu.VMEM((AGG_CHUNK, D), jnp.float32),
            pltpu.VMEM((AGG_CHUNK, D), jnp.float32),
            pltpu.VMEM_SHARED((NACC, D), jnp.float32),
            pltpu.SemaphoreType.DMA,
            pltpu.SemaphoreType.DMA,
        ],
        compiler_params=pltpu.CompilerParams(needs_layout_passes=False),
    )
    def call(hs_hbm, src_hbm, dst_hbm, out_hbm, sidx, didx, rows0, rows1,
             agg_sh, gsem0, gsem1):
        cid = lax.axis_index("c")
        sid = lax.axis_index("s")
        wid = cid * NS + sid

        def zl(i, _):
            for j in range(D // L):
                rows0[i, pl.ds(j * L, L)] = jnp.zeros((L,), jnp.float32)
            return 0
        lax.fori_loop(0, AGG_CHUNK, zl, 0)
        sbase = sid * ACC_PER_TILE
        for r in range(4):
            pltpu.sync_copy(rows0, agg_sh.at[pl.ds(sbase + r * 128, 128)])
        pltpu.sync_copy(rows0.at[pl.ds(0, ACC_PER_TILE - 512)],
                        agg_sh.at[pl.ds(sbase + 512, ACC_PER_TILE - 512)])
        plsc.subcore_barrier()

        dummy = hs_hbm.at[pl.ds(0, AGG_CHUNK)]
        rowbase = wid * AGG_NCHUNK

        for h in range(2):
            hb = h * AGG_HALF
            pltpu.sync_copy(src_hbm.at[pl.ds(rowbase + hb, AGG_HALF), :], sidx)
            pltpu.sync_copy(dst_hbm.at[pl.ds(rowbase + hb, AGG_HALF), :], didx)

            pltpu.async_copy(hs_hbm.at[sidx.at[0]], rows0, gsem0)

            def pair(g, _):
                k = 2 * g
                pltpu.make_async_copy(dummy, rows0, gsem0).wait()
                pltpu.async_copy(hs_hbm.at[sidx.at[k + 1]], rows1, gsem1)
                pltpu.sync_copy(rows0, agg_sh.at[didx.at[k]], add=True)

                pltpu.make_async_copy(dummy, rows1, gsem1).wait()
                pltpu.async_copy(hs_hbm.at[sidx.at[k + 2]], rows0, gsem0)
                pltpu.sync_copy(rows1, agg_sh.at[didx.at[k + 1]], add=True)
                return 0
            lax.fori_loop(0, (AGG_HALF - 2) // 2, pair, 0)

            k = AGG_HALF - 2
            pltpu.make_async_copy(dummy, rows0, gsem0).wait()
            pltpu.async_copy(hs_hbm.at[sidx.at[k + 1]], rows1, gsem1)
            pltpu.sync_copy(rows0, agg_sh.at[didx.at[k]], add=True)
            pltpu.make_async_copy(dummy, rows1, gsem1).wait()
            pltpu.sync_copy(rows1, agg_sh.at[didx.at[k + 1]], add=True)

        plsc.subcore_barrier()
        pltpu.sync_copy(agg_sh.at[pl.ds(sbase, ACC_PER_TILE)],
                        out_hbm.at[cid, pl.ds(sbase, ACC_PER_TILE)])

    return call(hs, src2d, dst2d)


_BLKP = 1024
_BLK = 1000
_GRID = 10


def _tc1_body(x_ref, dinv_ref, w_ref, o_ref):
    o_ref[...] = jnp.dot(x_ref[...] * dinv_ref[...], w_ref[...],
                         preferred_element_type=jnp.float32)


def _tc2_body(p0_ref, p1_ref, hs_ref, dinv_ref, b_ref, w_ref, o_ref):
    agg = (p0_ref[...] + p1_ref[...] + hs_ref[...]) * dinv_ref[...]
    x1 = jnp.maximum(agg + b_ref[...], 0.0)
    o_ref[...] = jnp.dot(x1 * dinv_ref[...], w_ref[...],
                         preferred_element_type=jnp.float32)


def _tc3_body(p0_ref, p1_ref, hs_ref, dinv_ref, b_ref, o_ref):
    o_ref[...] = ((p0_ref[...] + p1_ref[...] + hs_ref[...]) * dinv_ref[...]
                  + b_ref[...])


def _row_spec():
    return pl.BlockSpec((_BLK, D), lambda i: (i, 0))


def _rowp_spec():
    return pl.BlockSpec((_BLKP, D), lambda i: (i, 0))


def _full_spec():
    return pl.BlockSpec((D, D), lambda i: (0, 0))


def _bias_spec():
    return pl.BlockSpec((1, D), lambda i: (0, 0))


def _tc1(x, dinv_bc, w):
    return pl.pallas_call(
        _tc1_body,
        grid=(_GRID,),
        in_specs=[_rowp_spec(), _rowp_spec(), _full_spec()],
        out_specs=_rowp_spec(),
        out_shape=jax.ShapeDtypeStruct((NPAD, D), jnp.float32),
    )(x, dinv_bc, w)


def _tc2(p0, p1, hs, dinv_bc, b, w):
    return pl.pallas_call(
        _tc2_body,
        grid=(_GRID,),
        in_specs=[_rowp_spec(), _rowp_spec(), _rowp_spec(), _rowp_spec(),
                  _bias_spec(), _full_spec()],
        out_specs=_rowp_spec(),
        out_shape=jax.ShapeDtypeStruct((NPAD, D), jnp.float32),
    )(p0, p1, hs, dinv_bc, b, w)


def _tc3(p0, p1, hs, dinv_bc, b):
    return pl.pallas_call(
        _tc3_body,
        grid=(_GRID,),
        in_specs=[_row_spec(), _row_spec(), _row_spec(), _row_spec(),
                  _bias_spec()],
        out_specs=_row_spec(),
        out_shape=jax.ShapeDtypeStruct((N_GENES, D), jnp.float32),
    )(p0, p1, hs, dinv_bc, b)


def kernel(gene_ind_vec, edge_index, gene_embedding, W1, b1, W2, b2):
    src = edge_index[0]
    dst = edge_index[1]
    npad_e = EPAD - N_EDGES
    src2d = jnp.concatenate(
        [src, jnp.zeros((npad_e,), src.dtype)]).reshape(-1, AGG_CHUNK)
    dst2d = jnp.concatenate(
        [dst, jnp.full((npad_e,), TRASH_ROW, dst.dtype)]).reshape(-1, AGG_CHUNK)

    dinv_pad = _deg_call(dst)
    dinv_bc = jnp.broadcast_to(dinv_pad[:, None], (NPAD, D))
    x_pad = jnp.concatenate(
        [gene_embedding, jnp.zeros((NPAD - N_GENES, D), jnp.float32)])

    hs1 = _tc1(x_pad, dinv_bc, W1)
    agg1 = _agg_call(hs1, src2d, dst2d)
    hs2 = _tc2(agg1[0], agg1[1], hs1, dinv_bc, b1.reshape(1, D), W2)
    agg2 = _agg_call(hs2, src2d, dst2d)
    out = _tc3(agg2[0], agg2[1], hs2, dinv_bc, b2.reshape(1, D))
    return out
